# Initial kernel scaffold; baseline (speedup 1.0000x reference)
import jax
import jax.numpy as jnp
from jax.experimental import pallas as pl


def kernel(confOut, bboxOut, target, predBoxes):
    raise NotImplementedError("write your pallas kernel here")


# speedup vs baseline: 21.4359x; 21.4359x over previous
import jax
import jax.numpy as jnp
from jax import lax
from jax.experimental import pallas as pl

_CHUNK = 2000


def _prep_kernel(conf_ref, bbox_ref, pred_ref, aux_ref, tb_ref, bg_ref, scal_ref):
    j = pl.program_id(1)
    conf = conf_ref[0]
    n = conf.shape[0]
    m = tb_ref.shape[1]

    mx = jnp.max(conf, axis=1)
    ex = jnp.exp(conf - mx[:, None])
    mlse = mx + jnp.log(jnp.sum(ex, axis=1))
    bg = conf[:, 0] - mlse

    aux = aux_ref[0]
    num = aux[0, 0]
    ks = aux[0, 1:1 + m]
    cls = aux[0, 1 + m:1 + 2 * m]
    ksl = ks - j * n

    valid = (lax.broadcasted_iota(jnp.int32, (m, 1), 0)[:, 0] < num)
    inside = (ksl >= 0) & (ksl < n)
    vin = valid & inside
    pos = vin & (cls > 0)
    vinf = vin.astype(jnp.float32)
    posf = pos.astype(jnp.float32)

    onehot = (lax.broadcasted_iota(jnp.int32, (m, n), 1) == ksl[:, None]).astype(jnp.float32)

    rowvals = jnp.dot(onehot, conf, preferred_element_type=jnp.float32)
    c = conf.shape[1]
    clsoh = (lax.broadcasted_iota(jnp.int32, (m, c), 1) == cls[:, None]).astype(jnp.float32)
    confval = jnp.sum(rowvals * clsoh, axis=1)
    mlse_k = jnp.sum(onehot * mlse[None, :], axis=1)
    posSum = jnp.sum(jnp.where(pos, confval - mlse_k, jnp.float32(0.0)))
    pnum = jnp.sum(posf)

    anyhit = jnp.sum(onehot * vinf[:, None], axis=0)
    bg_ref[...] = jnp.where(anyhit > 0.0, jnp.float32(0.0), bg).reshape(1, 1, 1, n)

    bb_k = jnp.dot(onehot, bbox_ref[0], preferred_element_type=jnp.float32)
    pd_k = jnp.dot(onehot, pred_ref[...], preferred_element_type=jnp.float32)
    tb = tb_ref[0]
    pw = pd_k[:, 2] - pd_k[:, 0]
    ph = pd_k[:, 3] - pd_k[:, 1]
    pcx = (pd_k[:, 0] + pd_k[:, 2]) * 0.5
    pcy = (pd_k[:, 1] + pd_k[:, 3]) * 0.5
    tw = tb[:, 2] - tb[:, 0]
    th = tb[:, 3] - tb[:, 1]
    tcx = (tb[:, 0] + tb[:, 2]) * 0.5
    tcy = (tb[:, 1] + tb[:, 3]) * 0.5
    e0 = (tcx - pcx) / pw
    e1 = (tcy - pcy) / ph
    e2 = jnp.log(tw / pw)
    e3 = jnp.log(th / ph)
    enc = jnp.stack([e0, e1, e2, e3], axis=1)
    enc = jnp.where(vin[:, None], enc, jnp.float32(0.0))
    diff = jnp.abs(bb_k - enc)
    sl1 = jnp.where(diff < 1.0, 0.5 * diff * diff, diff - 0.5)
    slSum = jnp.sum(jnp.where(vin[:, None], sl1, jnp.float32(0.0)))
    cnt = jnp.sum(vinf)

    iota128 = lax.broadcasted_iota(jnp.int32, (1, 1, 128), 2)
    part = (posSum * (iota128 == 0) + pnum * (iota128 == 1)
            + slSum * (iota128 == 2) + cnt * (iota128 == 3)).astype(jnp.float32)

    @pl.when(j == 0)
    def _init():
        scal_ref[...] = part

    @pl.when(j != 0)
    def _acc():
        scal_ref[...] = scal_ref[...] + part


def _mine_kernel(bg_ref, scal_ref, out_ref):
    b2 = bg_ref[...]
    sc = scal_ref[...]
    col = lax.broadcasted_iota(jnp.int32, sc.shape, 1)
    posSum = jnp.sum(jnp.where(col == 0, sc, 0.0))
    pnum = jnp.sum(jnp.where(col == 1, sc, 0.0))
    slSum = jnp.sum(jnp.where(col == 2, sc, 0.0))
    cnt = jnp.sum(jnp.where(col == 3, sc, 0.0))

    v = jnp.float32(0.0) - b2
    key = lax.bitcast_convert_type(v, jnp.int32)
    K = (3.0 * pnum).astype(jnp.int32)

    def body(_, carry):
        lo, hi = carry
        mid = lo + ((hi - lo + 1) // 2)
        cge = jnp.sum((key >= mid).astype(jnp.int32))
        take = cge >= K
        return jnp.where(take, mid, lo), jnp.where(take, hi, mid - 1)

    lo, _ = lax.fori_loop(0, 32, body, (jnp.int32(0), jnp.int32(0x7F800000)))
    vK = lax.bitcast_convert_type(lo, jnp.float32)
    gt = key > lo
    cnt_gt = jnp.sum(gt.astype(jnp.int32))
    sum_gt = jnp.sum(jnp.where(gt, b2, 0.0))
    negSum = sum_gt + (K - cnt_gt).astype(jnp.float32) * (0.0 - vK)

    confLoss = -(posSum + negSum) / (4.0 * pnum)
    bboxLoss = slSum / (4.0 * cnt)
    iota128 = lax.broadcasted_iota(jnp.int32, (1, 128), 1)
    out_ref[...] = (confLoss * (iota128 == 0) + bboxLoss * (iota128 == 1)).astype(jnp.float32)


def kernel(confOut, bboxOut, target, predBoxes):
    b, n, c = confOut.shape
    m = (target.shape[1] - 1) // 6
    nc = n // _CHUNK

    num = target[:, 0].astype(jnp.int32)
    slots = target[:, 1:].reshape(b, m, 6)
    ks = slots[:, :, 5].astype(jnp.int32)
    cls = slots[:, :, 0].astype(jnp.int32)
    trueB = slots[:, :, 1:5]
    pad = jnp.zeros((b, 64 - 1 - 2 * m), jnp.int32)
    aux = jnp.concatenate([num[:, None], ks, cls, pad], axis=1).reshape(b, 1, 64)

    bg, scal = pl.pallas_call(
        _prep_kernel,
        grid=(b, nc),
        in_specs=[
            pl.BlockSpec((1, _CHUNK, c), lambda i, j: (i, j, 0)),
            pl.BlockSpec((1, _CHUNK, 4), lambda i, j: (i, j, 0)),
            pl.BlockSpec((_CHUNK, 4), lambda i, j: (j, 0)),
            pl.BlockSpec((1, 1, 64), lambda i, j: (i, 0, 0)),
            pl.BlockSpec((1, m, 4), lambda i, j: (i, 0, 0)),
        ],
        out_specs=[
            pl.BlockSpec((1, 1, 1, _CHUNK), lambda i, j: (i, j, 0, 0)),
            pl.BlockSpec((1, 1, 128), lambda i, j: (i, 0, 0)),
        ],
        out_shape=[
            jax.ShapeDtypeStruct((b, nc, 1, _CHUNK), jnp.float32),
            jax.ShapeDtypeStruct((b, 1, 128), jnp.float32),
        ],
    )(confOut, bboxOut, predBoxes, aux, trueB)

    out = pl.pallas_call(
        _mine_kernel,
        out_shape=jax.ShapeDtypeStruct((1, 128), jnp.float32),
    )(bg.reshape(b, n), scal.reshape(b, 128))

    return (out[0, 0], out[0, 1])


# speedup vs baseline: 50.0816x; 2.3363x over previous
import jax
import jax.numpy as jnp
from jax import lax
from jax.experimental import pallas as pl


def _prep_kernel(conf_ref, bbox_ref, pred_ref, aux_ref, tb_ref, bg_ref, scal_ref):
    conf = conf_ref[0]
    c, n = conf.shape
    m = tb_ref.shape[1]

    mx = jnp.max(conf, axis=0)
    ex = jnp.exp(conf - mx[None, :])
    mlse = mx + jnp.log(jnp.sum(ex, axis=0))
    bg = conf[0, :] - mlse

    aux = aux_ref[0]
    num = aux[0, 0]
    ks = aux[0, 1:1 + m]
    cls = aux[0, 1 + m:1 + 2 * m]

    valid = (lax.broadcasted_iota(jnp.int32, (m, 1), 0)[:, 0] < num)
    pos = valid & (cls > 0)
    vinf = valid.astype(jnp.float32)
    posf = pos.astype(jnp.float32)

    onehot = (lax.broadcasted_iota(jnp.int32, (m, n), 1) == ks[:, None]).astype(jnp.float32)

    rv = lax.dot_general(conf, onehot, (((1,), (1,)), ((), ())),
                         preferred_element_type=jnp.float32)
    clsohT = (lax.broadcasted_iota(jnp.int32, (c, m), 0) == cls[None, :]).astype(jnp.float32)
    confval = jnp.sum(rv * clsohT, axis=0)
    mlse_k = jnp.sum(onehot * mlse[None, :], axis=1)
    posSum = jnp.sum(jnp.where(pos, confval - mlse_k, jnp.float32(0.0)))
    pnum = jnp.sum(posf)

    anyhit = jnp.sum(onehot * vinf[:, None], axis=0)
    bg_ref[...] = jnp.where(anyhit > 0.0, jnp.float32(0.0), bg).reshape(1, 1, n)

    bb_k = jnp.dot(onehot, bbox_ref[0], preferred_element_type=jnp.float32)
    pd_k = jnp.dot(onehot, pred_ref[...], preferred_element_type=jnp.float32)
    tb = tb_ref[0]
    pw = pd_k[:, 2] - pd_k[:, 0]
    ph = pd_k[:, 3] - pd_k[:, 1]
    pcx = (pd_k[:, 0] + pd_k[:, 2]) * 0.5
    pcy = (pd_k[:, 1] + pd_k[:, 3]) * 0.5
    tw = tb[:, 2] - tb[:, 0]
    th = tb[:, 3] - tb[:, 1]
    tcx = (tb[:, 0] + tb[:, 2]) * 0.5
    tcy = (tb[:, 1] + tb[:, 3]) * 0.5
    e0 = (tcx - pcx) / pw
    e1 = (tcy - pcy) / ph
    e2 = jnp.log(tw / pw)
    e3 = jnp.log(th / ph)
    enc = jnp.stack([e0, e1, e2, e3], axis=1)
    enc = jnp.where(valid[:, None], enc, jnp.float32(0.0))
    diff = jnp.abs(bb_k - enc)
    sl1 = jnp.where(diff < 1.0, 0.5 * diff * diff, diff - 0.5)
    slSum = jnp.sum(jnp.where(valid[:, None], sl1, jnp.float32(0.0)))
    cnt = jnp.sum(vinf)

    iota128 = lax.broadcasted_iota(jnp.int32, (1, 1, 128), 2)
    scal_ref[...] = (posSum * (iota128 == 0) + pnum * (iota128 == 1)
                     + slSum * (iota128 == 2) + cnt * (iota128 == 3)).astype(jnp.float32)


def _mine_kernel(bg_ref, scal_ref, out_ref):
    b2 = bg_ref[...]
    sc = scal_ref[...]
    col = lax.broadcasted_iota(jnp.int32, sc.shape, 1)
    posSum = jnp.sum(jnp.where(col == 0, sc, 0.0))
    pnum = jnp.sum(jnp.where(col == 1, sc, 0.0))
    slSum = jnp.sum(jnp.where(col == 2, sc, 0.0))
    cnt = jnp.sum(jnp.where(col == 3, sc, 0.0))

    v = jnp.float32(0.0) - b2
    key = lax.bitcast_convert_type(v, jnp.int32)
    K = (3.0 * pnum).astype(jnp.int32)

    def body(_, carry):
        lo, hi = carry
        mid = lo + ((hi - lo + 1) // 2)
        cge = jnp.sum((key >= mid).astype(jnp.int32))
        take = cge >= K
        return jnp.where(take, mid, lo), jnp.where(take, hi, mid - 1)

    lo, _ = lax.fori_loop(0, 32, body, (jnp.int32(0), jnp.int32(0x7F800000)))
    vK = lax.bitcast_convert_type(lo, jnp.float32)
    gt = key > lo
    cnt_gt = jnp.sum(gt.astype(jnp.int32))
    sum_gt = jnp.sum(jnp.where(gt, b2, 0.0))
    negSum = sum_gt + (K - cnt_gt).astype(jnp.float32) * (0.0 - vK)

    confLoss = -(posSum + negSum) / (4.0 * pnum)
    bboxLoss = slSum / (4.0 * cnt)
    iota128 = lax.broadcasted_iota(jnp.int32, (1, 128), 1)
    out_ref[...] = (confLoss * (iota128 == 0) + bboxLoss * (iota128 == 1)).astype(jnp.float32)


def kernel(confOut, bboxOut, target, predBoxes):
    b, n, c = confOut.shape
    m = (target.shape[1] - 1) // 6

    num = target[:, 0].astype(jnp.int32)
    slots = target[:, 1:].reshape(b, m, 6)
    ks = slots[:, :, 5].astype(jnp.int32)
    cls = slots[:, :, 0].astype(jnp.int32)
    trueB = slots[:, :, 1:5]
    pad = jnp.zeros((b, 64 - 1 - 2 * m), jnp.int32)
    aux = jnp.concatenate([num[:, None], ks, cls, pad], axis=1).reshape(b, 1, 64)
    confT = jnp.transpose(confOut, (0, 2, 1))

    bg, scal = pl.pallas_call(
        _prep_kernel,
        grid=(b,),
        in_specs=[
            pl.BlockSpec((1, c, n), lambda i: (i, 0, 0)),
            pl.BlockSpec((1, n, 4), lambda i: (i, 0, 0)),
            pl.BlockSpec((n, 4), lambda i: (0, 0)),
            pl.BlockSpec((1, 1, 64), lambda i: (i, 0, 0)),
            pl.BlockSpec((1, m, 4), lambda i: (i, 0, 0)),
        ],
        out_specs=[
            pl.BlockSpec((1, 1, n), lambda i: (i, 0, 0)),
            pl.BlockSpec((1, 1, 128), lambda i: (i, 0, 0)),
        ],
        out_shape=[
            jax.ShapeDtypeStruct((b, 1, n), jnp.float32),
            jax.ShapeDtypeStruct((b, 1, 128), jnp.float32),
        ],
    )(confT, bboxOut, predBoxes, aux, trueB)

    out = pl.pallas_call(
        _mine_kernel,
        out_shape=jax.ShapeDtypeStruct((1, 128), jnp.float32),
    )(bg.reshape(b, n), scal.reshape(b, 128))

    return (out[0, 0], out[0, 1])
